# baseline (speedup 1.0000x reference)
import jax
import jax.numpy as jnp
from jax import lax
from jax.experimental import pallas as pl
from jax.experimental.pallas import tpu as pltpu
from jax.experimental.pallas import tpu_sc as plsc

_NC, _NS = 2, 16
_NW = _NC * _NS
_LANES = 16
_R = 16
_D = 1024
_TILE = _R * _D
_VECS = _TILE // _LANES


def _sc_body(x_hbm, e_hbm, o_hbm,
             bx0, bx1, ba0, ba1, ba2,
             sx0, sx1, sa0, sa1, sa2, so0, so1, so2):
    bx, ba = [bx0, bx1], [ba0, ba1, ba2]
    sx, sa, so = [sx0, sx1], [sa0, sa1, sa2], [so0, so1, so2]

    wid = lax.axis_index("s") * _NC + lax.axis_index("c")
    per_w = x_hbm.shape[0] // _NW
    nsteps = per_w // _TILE
    e_total = e_hbm.shape[0]
    x_base = wid * per_w
    e_base = lax.rem(x_base, e_total)

    hx = [None, None]
    ha = [None, None, None]
    ho = [None, None, None]

    for t in range(min(2, nsteps)):
        hx[t % 2] = pltpu.async_copy(
            x_hbm.at[pl.ds(x_base + t * _TILE, _TILE)], bx[t % 2], sx[t % 2])
        ha[t % 3] = pltpu.async_copy(
            e_hbm.at[pl.ds(e_base + t * _TILE, _TILE)], ba[t % 3], sa[t % 3])

    for t in range(nsteps):
        xc, ac = t % 2, t % 3
        hx[xc].wait()
        ha[ac].wait()
        bxc, bac = bx[xc], ba[ac]

        @plsc.parallel_loop(0, _VECS, unroll=8)
        def _add(i, bxc=bxc, bac=bac):
            s = pl.ds(i * _LANES, _LANES)
            plsc.addupdate(bac.at[s], bxc[s])

        ho[ac] = pltpu.async_copy(
            bac, o_hbm.at[pl.ds(x_base + t * _TILE, _TILE)], so[ac])

        nt = t + 2
        if nt < nsteps:
            nac = nt % 3
            if ho[nac] is not None:
                ho[nac].wait()
            ha[nac] = pltpu.async_copy(
                e_hbm.at[pl.ds(e_base + nt * _TILE, _TILE)], ba[nac], sa[nac])
            hx[xc] = pltpu.async_copy(
                x_hbm.at[pl.ds(x_base + nt * _TILE, _TILE)], bx[xc], sx[xc])

    for k in range(min(3, nsteps)):
        ho[(nsteps - 1 - k) % 3].wait()


def kernel(x, emb):
    B, S, D = x.shape
    n = B * S * D
    x1 = x.reshape(n)
    e1 = emb[:S].reshape(S * D)

    mesh = plsc.VectorSubcoreMesh(
        core_axis_name="c", subcore_axis_name="s",
        num_cores=_NC, num_subcores=_NS)
    run = pl.kernel(
        _sc_body,
        out_type=jax.ShapeDtypeStruct((n,), x.dtype),
        mesh=mesh,
        scratch_types=(
            [pltpu.VMEM((_TILE,), jnp.float32)] * 2
            + [pltpu.VMEM((_TILE,), jnp.float32)] * 3
            + [pltpu.SemaphoreType.DMA] * 8
        ),
    )
    return run(x1, e1).reshape(B, S, D)


# speedup vs baseline: 2.6141x; 2.6141x over previous
import jax
import jax.numpy as jnp
from jax import lax
from jax.experimental import pallas as pl
from jax.experimental.pallas import tpu as pltpu
from jax.experimental.pallas import tpu_sc as plsc

_NC, _NS = 2, 16
_NW = _NC * _NS
_LANES = 16
_R = 16
_D = 1024


def _sc_body(x_hbm, e_hbm, o_hbm, bx0, bx1, bx2, be0, be1,
             sx0, sx1, sx2, se0, se1, so0, so1, so2):
    bx, be = [bx0, bx1, bx2], [be0, be1]
    sx, se, so = [sx0, sx1, sx2], [se0, se1], [so0, so1, so2]

    wid = lax.axis_index("s") * _NC + lax.axis_index("c")
    s_total, _ = e_hbm.shape
    rows_w = s_total // _NW
    nsteps = rows_w // _R
    nb = x_hbm.shape[0] // s_total
    npass = nsteps * nb
    s_base = wid * rows_w

    def x_row(p):
        t, b = p // nb, p % nb
        return b * s_total + s_base + t * _R

    hx = [None, None, None]
    he = [None, None]
    ho = [None, None, None]

    he[0] = pltpu.async_copy(e_hbm.at[pl.ds(s_base, _R)], be[0], se[0])
    for p in range(min(2, npass)):
        hx[p % 3] = pltpu.async_copy(
            x_hbm.at[pl.ds(x_row(p), _R)], bx[p % 3], sx[p % 3])

    for p in range(npass):
        xs, es = p % 3, (p // nb) % 2
        hx[xs].wait()
        if p % nb == 0:
            he[es].wait()
        bxc, bec = bx[xs], be[es]

        @plsc.parallel_loop(0, _R * (_D // _LANES), unroll=8)
        def _chunk(i, bxc=bxc, bec=bec):
            r = i >> 6
            s = pl.ds((i & 63) * _LANES, _LANES)
            plsc.addupdate(bxc.at[r, s], bec[r, s])

        ho[xs] = pltpu.async_copy(
            bxc, o_hbm.at[pl.ds(x_row(p), _R)], so[xs])

        np_ = p + 2
        if np_ < npass:
            nxs = np_ % 3
            if ho[nxs] is not None:
                ho[nxs].wait()
            hx[nxs] = pltpu.async_copy(
                x_hbm.at[pl.ds(x_row(np_), _R)], bx[nxs], sx[nxs])
            if np_ % nb == 0:
                nes = (np_ // nb) % 2
                he[nes] = pltpu.async_copy(
                    e_hbm.at[pl.ds(s_base + (np_ // nb) * _R, _R)],
                    be[nes], se[nes])

    for k in range(min(3, npass)):
        ho[(npass - 1 - k) % 3].wait()


def kernel(x, emb):
    B, S, D = x.shape
    x2 = x.reshape(B * S, D)
    e2 = emb[:S]

    mesh = plsc.VectorSubcoreMesh(
        core_axis_name="c", subcore_axis_name="s",
        num_cores=_NC, num_subcores=_NS)
    run = pl.kernel(
        _sc_body,
        out_type=jax.ShapeDtypeStruct((B * S, D), x.dtype),
        mesh=mesh,
        scratch_types=(
            [pltpu.VMEM((_R, _D), jnp.float32)] * 3
            + [pltpu.VMEM((_R, _D), jnp.float32)] * 2
            + [pltpu.SemaphoreType.DMA] * 8
        ),
        compiler_params=pltpu.CompilerParams(use_tc_tiling_on_sc=True),
    )
    return run(x2, e2).reshape(B, S, D)


# speedup vs baseline: 5.6627x; 2.1662x over previous
import jax
import jax.numpy as jnp
from jax.experimental import pallas as pl
from jax.experimental.pallas import tpu as pltpu


def _add_kernel(x_ref, e_ref, o_ref):
    o_ref[...] = x_ref[...] + e_ref[...][None, :, :]


def kernel(x, emb):
    B, S, D = x.shape
    return pl.pallas_call(
        _add_kernel,
        grid=(B,),
        in_specs=[
            pl.BlockSpec((1, S, D), lambda i: (i, 0, 0)),
            pl.BlockSpec((S, D), lambda i: (0, 0)),
        ],
        out_specs=pl.BlockSpec((1, S, D), lambda i: (i, 0, 0)),
        out_shape=jax.ShapeDtypeStruct((B, S, D), x.dtype),
        compiler_params=pltpu.CompilerParams(
            vmem_limit_bytes=100 * 1024 * 1024),
    )(x, emb)
